# baseline (speedup 1.0000x reference)
import functools
import math

import jax
import jax.numpy as jnp
from jax import lax
from jax.experimental import pallas as pl
from jax.experimental.pallas import tpu as pltpu
from jax.experimental.pallas import tpu_sc as plsc

N = 50000
E = 800000
D_IN = 128
H = 64
OUT = 256
BN_EPS = 1e-5

NC = 2
NS = 16
CHUNK = 256
G = 4
CPG = G
NBUF = 2
NG = 50
NCHUNK = G * NG
EPT = CHUNK * NCHUNK
EPAD = EPT * NS
ZROWS = 3128
ACC_ROWS = ZROWS * NS
OPT = 3128
NPAD = OPT * NS

BN = 2000
GRID = N // BN

_HIGH = jax.lax.Precision.HIGHEST


def _sc_agg_body(hflat_hbm, src_hbm, dst_hbm, z_hbm, out_hbm,
                 acc, si0, si1, di0, di1, rows0, rows1,
                 sis0, sis1, dis0, dis1, gs0, gs1, scs0, scs1):
    c = lax.axis_index("c")
    s = lax.axis_index("s")
    si = (si0, si1)
    di = (di0, di1)
    rows = (rows0, rows1)
    sisem = (sis0, sis1)
    disem = (dis0, dis1)
    gsem = (gs0, gs1)
    ssem = (scs0, scs1)

    pltpu.sync_copy(z_hbm, acc.at[pl.ds(s * ZROWS, ZROWS)])

    def _idx_start(g, p):
        g = jnp.minimum(g, NG - 1)
        pltpu.async_copy(src_hbm.at[c, s, pl.ds(g * G, G)], si[p], sisem[p])
        pltpu.async_copy(dst_hbm.at[s, pl.ds(g * G, G)], di[p], disem[p])

    def _idx_wait(p):
        pltpu.make_async_copy(src_hbm.at[c, s, pl.ds(0, G)], si[p],
                              sisem[p]).wait()
        pltpu.make_async_copy(dst_hbm.at[s, pl.ds(0, G)], di[p],
                              disem[p]).wait()

    def _gather_start(p, j, b):
        pltpu.async_copy(hflat_hbm.at[si[p].at[j]], rows[b], gsem[b])

    def _gather_wait(b):
        pltpu.make_async_copy(hflat_hbm.at[si[0].at[0]], rows[b],
                              gsem[b]).wait()

    def _scatter_start(p, j, b):
        pltpu.async_copy(rows[b], acc.at[di[p].at[j]], ssem[b], add=True)

    def _scatter_wait(b):
        pltpu.make_async_copy(rows[b], acc.at[di[0].at[0]], ssem[b]).wait()

    def _block(g0):
        for p in range(2):
            for j in range(CPG):
                b = (p * CPG + j) % NBUF
                bn = (p * CPG + j + 2) % NBUF
                _gather_wait(b)
                _scatter_start(p, j, b)
                _scatter_wait(b)
                nj = j + 2
                if nj < CPG:
                    _gather_start(p, nj, bn)
                else:
                    if nj == CPG:
                        _idx_wait(p ^ 1)
                    _gather_start(p ^ 1, nj - CPG, bn)
            _idx_start(g0 + p + 2, p)

    _idx_start(jnp.int32(0), 0)
    _idx_start(jnp.int32(1), 1)
    plsc.subcore_barrier()
    _idx_wait(0)
    _gather_start(0, 0, 0)
    _gather_start(0, 1, 1)

    def _outer(gg, carry):
        _block(gg * 2)
        return carry

    lax.fori_loop(0, NG // 2, _outer, 0)
    _gather_wait(NCHUNK % NBUF)
    _gather_wait((NCHUNK + 1) % NBUF)
    _idx_wait(1)

    plsc.subcore_barrier()
    pltpu.sync_copy(acc.at[pl.ds(s * OPT, OPT)],
                    out_hbm.at[pl.ds(s * OPT, OPT), c, :])


def _sc_agg(hflat, src2, dst3, zblk):
    mesh = plsc.VectorSubcoreMesh(core_axis_name="c", subcore_axis_name="s")
    return pl.kernel(
        _sc_agg_body,
        out_type=jax.ShapeDtypeStruct((NPAD, NC, 32), jnp.float32),
        mesh=mesh,
        scratch_types=[
            pltpu.VMEM_SHARED((ACC_ROWS, 32), jnp.float32),
            pltpu.VMEM((G, CHUNK), jnp.int32),
            pltpu.VMEM((G, CHUNK), jnp.int32),
            pltpu.VMEM((G, CHUNK), jnp.int32),
            pltpu.VMEM((G, CHUNK), jnp.int32),
            pltpu.VMEM((CHUNK, 32), jnp.float32),
            pltpu.VMEM((CHUNK, 32), jnp.float32),
        ] + [pltpu.SemaphoreType.DMA] * 8,
        compiler_params=pltpu.CompilerParams(use_tc_tiling_on_sc=False),
        name="gin_sc_agg",
    )(hflat, src2, dst3, zblk)


def _tc_input_body(x_ref, w_ref, b_ref, out_ref):
    out_ref[...] = jnp.dot(x_ref[...], w_ref[...],
                           preferred_element_type=jnp.float32,
                           precision=_HIGH) + b_ref[...]


def _tc_input(x, w, b):
    return pl.pallas_call(
        _tc_input_body,
        grid=(GRID,),
        in_specs=[
            pl.BlockSpec((BN, D_IN), lambda i: (i, 0)),
            pl.BlockSpec((D_IN, H), lambda i: (0, 0)),
            pl.BlockSpec((1, H), lambda i: (0, 0)),
        ],
        out_specs=pl.BlockSpec((BN, H), lambda i: (i, 0)),
        out_shape=jax.ShapeDtypeStruct((N, H), jnp.float32),
    )(x, w, b)


def _mlp_block(h_ref, a_ref, wa_ref, ba_ref, wb_ref, bb_ref):
    t = 2.0 * h_ref[...] + a_ref[...]
    t = jnp.maximum(jnp.dot(t, wa_ref[...], preferred_element_type=jnp.float32,
                            precision=_HIGH) + ba_ref[...], 0.0)
    return jnp.maximum(jnp.dot(t, wb_ref[...],
                               preferred_element_type=jnp.float32,
                               precision=_HIGH) + bb_ref[...], 0.0)


def _tc_layer_body(h_ref, a_ref, wa_ref, ba_ref, wb_ref, bb_ref, out_ref):
    out_ref[...] = _mlp_block(h_ref, a_ref, wa_ref, ba_ref, wb_ref, bb_ref)


_LAYER_IN_SPECS = [
    pl.BlockSpec((BN, H), lambda i: (i, 0)),
    pl.BlockSpec((BN, H), lambda i: (i, 0)),
    pl.BlockSpec((H, H), lambda i: (0, 0)),
    pl.BlockSpec((1, H), lambda i: (0, 0)),
    pl.BlockSpec((H, H), lambda i: (0, 0)),
    pl.BlockSpec((1, H), lambda i: (0, 0)),
]


def _tc_layer(h, agg, wa, ba, wb, bb):
    return pl.pallas_call(
        _tc_layer_body,
        grid=(GRID,),
        in_specs=_LAYER_IN_SPECS,
        out_specs=pl.BlockSpec((BN, H), lambda i: (i, 0)),
        out_shape=jax.ShapeDtypeStruct((N, H), jnp.float32),
    )(h, agg, wa, ba, wb, bb)


def _tc_last_body(h_ref, a_ref, wa_ref, ba_ref, wb_ref, bb_ref,
                  out_ref, sum_ref):
    i = pl.program_id(0)
    t = _mlp_block(h_ref, a_ref, wa_ref, ba_ref, wb_ref, bb_ref)
    out_ref[...] = t
    part = jnp.sum(t, axis=0, keepdims=True)

    @pl.when(i == 0)
    def _init():
        sum_ref[...] = part

    @pl.when(i != 0)
    def _acc():
        sum_ref[...] += part


def _tc_last(h, agg, wa, ba, wb, bb):
    return pl.pallas_call(
        _tc_last_body,
        grid=(GRID,),
        in_specs=_LAYER_IN_SPECS,
        out_specs=[
            pl.BlockSpec((BN, H), lambda i: (i, 0)),
            pl.BlockSpec((1, H), lambda i: (0, 0)),
        ],
        out_shape=[
            jax.ShapeDtypeStruct((N, H), jnp.float32),
            jax.ShapeDtypeStruct((1, H), jnp.float32),
        ],
    )(h, agg, wa, ba, wb, bb)


def _tc_head_body(s_ref, w1_ref, b1_ref, w2_ref, b2_ref, out_ref):
    g = s_ref[...] * (1.0 / N)
    z = jnp.maximum(jnp.dot(g, w1_ref[...], preferred_element_type=jnp.float32,
                            precision=_HIGH) + b1_ref[...], 0.0)
    out_ref[...] = jnp.dot(z, w2_ref[...], preferred_element_type=jnp.float32,
                           precision=_HIGH) + b2_ref[...]


def _tc_head(hsum, w1, b1, w2, b2):
    return pl.pallas_call(
        _tc_head_body,
        out_shape=jax.ShapeDtypeStruct((1, OUT), jnp.float32),
    )(hsum, w1, b1, w2, b2)


def _fold_bn(w, b, g, be):
    gs = g * (1.0 / math.sqrt(1.0 + BN_EPS))
    return w * gs[None, :], (b * gs + be)[None, :]


def kernel(x, edge_index, W_in, b_in,
           W0_1, b0_1, g0_1, be0_1, W0_2, b0_2, g0_2, be0_2,
           W1_1, b1_1, g1_1, be1_1, W1_2, b1_2, g1_2, be1_2,
           W2_1, b2_1, g2_1, be2_1, W2_2, b2_2, g2_2, be2_2,
           Wf1, bf1, gf, bef, Wf2, bf2):
    src = edge_index[0]
    dst = edge_index[1]
    pad = EPAD - E
    srcp = jnp.concatenate([src, jnp.zeros((pad,), jnp.int32)])
    src2 = jnp.stack([2 * srcp, 2 * srcp + 1]).reshape(NC, NS, NCHUNK, CHUNK)
    dst3 = jnp.concatenate(
        [dst, jnp.full((pad,), N, jnp.int32)]).reshape(NS, NCHUNK, CHUNK)
    zblk = jnp.zeros((ZROWS, 32), jnp.float32)

    layers = [
        _fold_bn(W0_1, b0_1, g0_1, be0_1) + _fold_bn(W0_2, b0_2, g0_2, be0_2),
        _fold_bn(W1_1, b1_1, g1_1, be1_1) + _fold_bn(W1_2, b1_2, g1_2, be1_2),
        _fold_bn(W2_1, b2_1, g2_1, be2_1) + _fold_bn(W2_2, b2_2, g2_2, be2_2),
    ]
    wf1, bf1f = _fold_bn(Wf1, bf1, gf, bef)

    h = _tc_input(x, W_in, b_in[None, :])
    hsum = None
    for li, (wa, ba, wb, bb) in enumerate(layers):
        agg = _sc_agg(h.reshape(NC * N, 32), src2, dst3, zblk)
        agg = agg.reshape(NPAD, NC * 32)
        if li < 2:
            h = _tc_layer(h, agg, wa, ba, wb, bb)
        else:
            h, hsum = _tc_last(h, agg, wa, ba, wb, bb)
    return _tc_head(hsum, wf1, bf1f, Wf2, bf2[None, :])


# speedup vs baseline: 1.4507x; 1.4507x over previous
import functools
import math

import jax
import jax.numpy as jnp
from jax import lax
from jax.experimental import pallas as pl
from jax.experimental.pallas import tpu as pltpu
from jax.experimental.pallas import tpu_sc as plsc

N = 50000
E = 800000
D_IN = 128
H = 64
OUT = 256
BN_EPS = 1e-5

NC = 2
NS = 16
CHUNK = 256
G = 4
CPG = G
NBUF = 2
NG = 50
NCHUNK = G * NG
EPT = CHUNK * NCHUNK
EPAD = EPT * NS
ZROWS = 3128
ACC_ROWS = ZROWS * NS
OPT = 3128
NPAD = OPT * NS

BN = 2000
GRID = N // BN

_HIGH = jax.lax.Precision.HIGHEST


def _sc_agg_body(hflat_hbm, src_hbm, dst_hbm, z_hbm, out_hbm,
                 acc, si0, si1, di0, di1, rows0, rows1,
                 sis0, sis1, dis0, dis1, gs0, gs1, scs0, scs1):
    c = lax.axis_index("c")
    s = lax.axis_index("s")
    si = (si0, si1)
    di = (di0, di1)
    rows = (rows0, rows1)
    sisem = (sis0, sis1)
    disem = (dis0, dis1)
    gsem = (gs0, gs1)
    ssem = (scs0, scs1)

    pltpu.sync_copy(z_hbm, acc.at[pl.ds(s * ZROWS, ZROWS)])

    def _idx_start(g, p):
        g = jnp.minimum(g, NG - 1)
        pltpu.async_copy(src_hbm.at[c, s, pl.ds(g * G, G)], si[p], sisem[p])
        pltpu.async_copy(dst_hbm.at[s, pl.ds(g * G, G)], di[p], disem[p])

    def _idx_wait(p):
        pltpu.make_async_copy(src_hbm.at[c, s, pl.ds(0, G)], si[p],
                              sisem[p]).wait()
        pltpu.make_async_copy(dst_hbm.at[s, pl.ds(0, G)], di[p],
                              disem[p]).wait()

    def _gather_start(p, j, b):
        pltpu.async_copy(hflat_hbm.at[si[p].at[j]], rows[b], gsem[b])

    def _gather_wait(b):
        pltpu.make_async_copy(hflat_hbm.at[si[0].at[0]], rows[b],
                              gsem[b]).wait()

    def _scatter_start(p, j, b):
        pltpu.async_copy(rows[b], acc.at[di[p].at[j]], ssem[b], add=True)

    def _scatter_wait(b):
        pltpu.make_async_copy(rows[b], acc.at[di[0].at[0]], ssem[b]).wait()

    def _block(g0):
        for p in range(2):
            for j in range(CPG):
                b = (p * CPG + j) % NBUF
                bn = (p * CPG + j + 2) % NBUF
                _gather_wait(b)
                _scatter_start(p, j, b)
                _scatter_wait(b)
                nj = j + 2
                if nj < CPG:
                    _gather_start(p, nj, bn)
                else:
                    if nj == CPG:
                        _idx_wait(p ^ 1)
                    _gather_start(p ^ 1, nj - CPG, bn)
            _idx_start(g0 + p + 2, p)

    _idx_start(jnp.int32(0), 0)
    _idx_start(jnp.int32(1), 1)
    plsc.subcore_barrier()
    _idx_wait(0)
    _gather_start(0, 0, 0)
    _gather_start(0, 1, 1)

    def _outer(gg, carry):
        _block(gg * 2)
        return carry

    lax.fori_loop(0, NG // 2, _outer, 0)
    _gather_wait(NCHUNK % NBUF)
    _gather_wait((NCHUNK + 1) % NBUF)
    _idx_wait(1)

    plsc.subcore_barrier()
    pltpu.sync_copy(acc.at[pl.ds(s * OPT, OPT)],
                    out_hbm.at[c, pl.ds(s * OPT, OPT)])


def _sc_agg(hflat, src2, dst3, zblk):
    mesh = plsc.VectorSubcoreMesh(core_axis_name="c", subcore_axis_name="s")
    return pl.kernel(
        _sc_agg_body,
        out_type=jax.ShapeDtypeStruct((NC, NPAD, 32), jnp.float32),
        mesh=mesh,
        scratch_types=[
            pltpu.VMEM_SHARED((ACC_ROWS, 32), jnp.float32),
            pltpu.VMEM((G, CHUNK), jnp.int32),
            pltpu.VMEM((G, CHUNK), jnp.int32),
            pltpu.VMEM((G, CHUNK), jnp.int32),
            pltpu.VMEM((G, CHUNK), jnp.int32),
            pltpu.VMEM((CHUNK, 32), jnp.float32),
            pltpu.VMEM((CHUNK, 32), jnp.float32),
        ] + [pltpu.SemaphoreType.DMA] * 8,
        compiler_params=pltpu.CompilerParams(use_tc_tiling_on_sc=False),
        name="gin_sc_agg",
    )(hflat, src2, dst3, zblk)


def _tc_input_body(x_ref, w_ref, b_ref, out_ref):
    h = jnp.dot(x_ref[...], w_ref[...], preferred_element_type=jnp.float32,
                precision=_HIGH) + b_ref[...]
    out_ref[0] = h[:, :32]
    out_ref[1] = h[:, 32:]


def _tc_input(x, w, b):
    return pl.pallas_call(
        _tc_input_body,
        grid=(GRID,),
        in_specs=[
            pl.BlockSpec((BN, D_IN), lambda i: (i, 0)),
            pl.BlockSpec((D_IN, H), lambda i: (0, 0)),
            pl.BlockSpec((1, H), lambda i: (0, 0)),
        ],
        out_specs=pl.BlockSpec((NC, BN, 32), lambda i: (0, i, 0)),
        out_shape=jax.ShapeDtypeStruct((NC, N, 32), jnp.float32),
    )(x, w, b)


def _mlp_block(h_ref, a_ref, wa_ref, ba_ref, wb_ref, bb_ref):
    h = jnp.concatenate([h_ref[0], h_ref[1]], axis=1)
    a = jnp.concatenate([a_ref[0], a_ref[1]], axis=1)
    t = 2.0 * h + a
    t = jnp.maximum(jnp.dot(t, wa_ref[...], preferred_element_type=jnp.float32,
                            precision=_HIGH) + ba_ref[...], 0.0)
    return jnp.maximum(jnp.dot(t, wb_ref[...],
                               preferred_element_type=jnp.float32,
                               precision=_HIGH) + bb_ref[...], 0.0)


def _tc_layer_body(h_ref, a_ref, wa_ref, ba_ref, wb_ref, bb_ref, out_ref):
    t = _mlp_block(h_ref, a_ref, wa_ref, ba_ref, wb_ref, bb_ref)
    out_ref[0] = t[:, :32]
    out_ref[1] = t[:, 32:]


_LAYER_IN_SPECS = [
    pl.BlockSpec((NC, BN, 32), lambda i: (0, i, 0)),
    pl.BlockSpec((NC, BN, 32), lambda i: (0, i, 0)),
    pl.BlockSpec((H, H), lambda i: (0, 0)),
    pl.BlockSpec((1, H), lambda i: (0, 0)),
    pl.BlockSpec((H, H), lambda i: (0, 0)),
    pl.BlockSpec((1, H), lambda i: (0, 0)),
]


def _tc_layer(h, agg, wa, ba, wb, bb):
    return pl.pallas_call(
        _tc_layer_body,
        grid=(GRID,),
        in_specs=_LAYER_IN_SPECS,
        out_specs=pl.BlockSpec((NC, BN, 32), lambda i: (0, i, 0)),
        out_shape=jax.ShapeDtypeStruct((NC, N, 32), jnp.float32),
    )(h, agg, wa, ba, wb, bb)


def _tc_last_body(h_ref, a_ref, wa_ref, ba_ref, wb_ref, bb_ref,
                  out_ref, sum_ref):
    i = pl.program_id(0)
    t = _mlp_block(h_ref, a_ref, wa_ref, ba_ref, wb_ref, bb_ref)
    out_ref[0] = t[:, :32]
    out_ref[1] = t[:, 32:]
    part = jnp.sum(t, axis=0, keepdims=True)

    @pl.when(i == 0)
    def _init():
        sum_ref[...] = part

    @pl.when(i != 0)
    def _acc():
        sum_ref[...] += part


def _tc_last(h, agg, wa, ba, wb, bb):
    return pl.pallas_call(
        _tc_last_body,
        grid=(GRID,),
        in_specs=_LAYER_IN_SPECS,
        out_specs=[
            pl.BlockSpec((NC, BN, 32), lambda i: (0, i, 0)),
            pl.BlockSpec((1, H), lambda i: (0, 0)),
        ],
        out_shape=[
            jax.ShapeDtypeStruct((NC, N, 32), jnp.float32),
            jax.ShapeDtypeStruct((1, H), jnp.float32),
        ],
    )(h, agg, wa, ba, wb, bb)


def _tc_head_body(s_ref, w1_ref, b1_ref, w2_ref, b2_ref, out_ref):
    g = s_ref[...] * (1.0 / N)
    z = jnp.maximum(jnp.dot(g, w1_ref[...], preferred_element_type=jnp.float32,
                            precision=_HIGH) + b1_ref[...], 0.0)
    out_ref[...] = jnp.dot(z, w2_ref[...], preferred_element_type=jnp.float32,
                           precision=_HIGH) + b2_ref[...]


def _tc_head(hsum, w1, b1, w2, b2):
    return pl.pallas_call(
        _tc_head_body,
        out_shape=jax.ShapeDtypeStruct((1, OUT), jnp.float32),
    )(hsum, w1, b1, w2, b2)


def _fold_bn(w, b, g, be):
    gs = g * (1.0 / math.sqrt(1.0 + BN_EPS))
    return w * gs[None, :], (b * gs + be)[None, :]


def kernel(x, edge_index, W_in, b_in,
           W0_1, b0_1, g0_1, be0_1, W0_2, b0_2, g0_2, be0_2,
           W1_1, b1_1, g1_1, be1_1, W1_2, b1_2, g1_2, be1_2,
           W2_1, b2_1, g2_1, be2_1, W2_2, b2_2, g2_2, be2_2,
           Wf1, bf1, gf, bef, Wf2, bf2):
    src = edge_index[0]
    dst = edge_index[1]
    pad = EPAD - E
    srcp = jnp.concatenate([src, jnp.zeros((pad,), jnp.int32)])
    src2 = jnp.stack([srcp, srcp + N]).reshape(NC, NS, NCHUNK, CHUNK)
    dst3 = jnp.concatenate(
        [dst, jnp.full((pad,), N, jnp.int32)]).reshape(NS, NCHUNK, CHUNK)
    zblk = jnp.zeros((ZROWS, 32), jnp.float32)

    layers = [
        _fold_bn(W0_1, b0_1, g0_1, be0_1) + _fold_bn(W0_2, b0_2, g0_2, be0_2),
        _fold_bn(W1_1, b1_1, g1_1, be1_1) + _fold_bn(W1_2, b1_2, g1_2, be1_2),
        _fold_bn(W2_1, b2_1, g2_1, be2_1) + _fold_bn(W2_2, b2_2, g2_2, be2_2),
    ]
    wf1, bf1f = _fold_bn(Wf1, bf1, gf, bef)

    h = _tc_input(x, W_in, b_in[None, :])
    hsum = None
    for li, (wa, ba, wb, bb) in enumerate(layers):
        agg = _sc_agg(h.reshape(NC * N, 32), src2, dst3, zblk)
        if li < 2:
            h = _tc_layer(h, agg, wa, ba, wb, bb)
        else:
            h, hsum = _tc_last(h, agg, wa, ba, wb, bb)
    return _tc_head(hsum, wf1, bf1f, Wf2, bf2[None, :])


# speedup vs baseline: 1.4627x; 1.0083x over previous
import functools
import math

import jax
import jax.numpy as jnp
from jax import lax
from jax.experimental import pallas as pl
from jax.experimental.pallas import tpu as pltpu
from jax.experimental.pallas import tpu_sc as plsc

N = 50000
E = 800000
D_IN = 128
H = 64
OUT = 256
BN_EPS = 1e-5

NC = 2
NS = 16
CHUNK = 256
G = 4
CPG = G
NBUF = 2
NG = 50
NCHUNK = G * NG
EPT = CHUNK * NCHUNK
EPAD = EPT * NS
ZROWS = 3128
ACC_ROWS = ZROWS * NS
OPT = 3128
NPAD = OPT * NS

BN = 2000
GRID = N // BN

_HIGH = jax.lax.Precision.HIGHEST


def _sc_agg_body(hflat_hbm, src_hbm, dst_hbm, out_hbm,
                 acc, si0, si1, di0, di1, rows0, rows1,
                 sis0, sis1, dis0, dis1, gs0, gs1, scs0, scs1):
    c = lax.axis_index("c")
    s = lax.axis_index("s")
    si = (si0, si1)
    di = (di0, di1)
    rows = (rows0, rows1)
    sisem = (sis0, sis1)
    disem = (dis0, dis1)
    gsem = (gs0, gs1)
    ssem = (scs0, scs1)

    z16 = jnp.zeros((16,), jnp.float32)

    def _zrow(r, carry):
        rows0[r, pl.ds(0, 16)] = z16
        rows0[r, pl.ds(16, 16)] = z16
        return carry

    lax.fori_loop(0, CHUNK, _zrow, 0)
    base = s * ZROWS
    for k in range(ZROWS // CHUNK):
        pltpu.sync_copy(rows0, acc.at[pl.ds(base + k * CHUNK, CHUNK)])
    tail = ZROWS % CHUNK
    pltpu.sync_copy(rows0.at[pl.ds(0, tail)],
                    acc.at[pl.ds(base + ZROWS - tail, tail)])

    def _idx_start(g, p):
        g = jnp.minimum(g, NG - 1)
        pltpu.async_copy(src_hbm.at[c, s, pl.ds(g * G, G)], si[p], sisem[p])
        pltpu.async_copy(dst_hbm.at[s, pl.ds(g * G, G)], di[p], disem[p])

    def _idx_wait(p):
        pltpu.make_async_copy(src_hbm.at[c, s, pl.ds(0, G)], si[p],
                              sisem[p]).wait()
        pltpu.make_async_copy(dst_hbm.at[s, pl.ds(0, G)], di[p],
                              disem[p]).wait()

    def _gather_start(p, j, b):
        pltpu.async_copy(hflat_hbm.at[si[p].at[j]], rows[b], gsem[b])

    def _gather_wait(b):
        pltpu.make_async_copy(hflat_hbm.at[si[0].at[0]], rows[b],
                              gsem[b]).wait()

    def _scatter_start(p, j, b):
        pltpu.async_copy(rows[b], acc.at[di[p].at[j]], ssem[b], add=True)

    def _scatter_wait(b):
        pltpu.make_async_copy(rows[b], acc.at[di[0].at[0]], ssem[b]).wait()

    def _block(g0):
        for p in range(2):
            for j in range(CPG):
                b = (p * CPG + j) % NBUF
                bn = (p * CPG + j + 2) % NBUF
                _gather_wait(b)
                _scatter_start(p, j, b)
                _scatter_wait(b)
                nj = j + 2
                if nj < CPG:
                    _gather_start(p, nj, bn)
                else:
                    if nj == CPG:
                        _idx_wait(p ^ 1)
                    _gather_start(p ^ 1, nj - CPG, bn)
            _idx_start(g0 + p + 2, p)

    _idx_start(jnp.int32(0), 0)
    _idx_start(jnp.int32(1), 1)
    plsc.subcore_barrier()
    _idx_wait(0)
    _gather_start(0, 0, 0)
    _gather_start(0, 1, 1)

    def _outer(gg, carry):
        _block(gg * 2)
        return carry

    lax.fori_loop(0, NG // 2, _outer, 0)
    _gather_wait(NCHUNK % NBUF)
    _gather_wait((NCHUNK + 1) % NBUF)
    _idx_wait(1)

    plsc.subcore_barrier()
    pltpu.sync_copy(acc.at[pl.ds(s * OPT, OPT)],
                    out_hbm.at[c, pl.ds(s * OPT, OPT)])


def _sc_agg(hflat, src2, dst3):
    mesh = plsc.VectorSubcoreMesh(core_axis_name="c", subcore_axis_name="s")
    return pl.kernel(
        _sc_agg_body,
        out_type=jax.ShapeDtypeStruct((NC, NPAD, 32), jnp.float32),
        mesh=mesh,
        scratch_types=[
            pltpu.VMEM_SHARED((ACC_ROWS, 32), jnp.float32),
            pltpu.VMEM((G, CHUNK), jnp.int32),
            pltpu.VMEM((G, CHUNK), jnp.int32),
            pltpu.VMEM((G, CHUNK), jnp.int32),
            pltpu.VMEM((G, CHUNK), jnp.int32),
            pltpu.VMEM((CHUNK, 32), jnp.float32),
            pltpu.VMEM((CHUNK, 32), jnp.float32),
        ] + [pltpu.SemaphoreType.DMA] * 8,
        compiler_params=pltpu.CompilerParams(use_tc_tiling_on_sc=False),
        name="gin_sc_agg",
    )(hflat, src2, dst3)


def _tc_input_body(x_ref, w_ref, b_ref, out_ref):
    h = jnp.dot(x_ref[...], w_ref[...], preferred_element_type=jnp.float32,
                precision=_HIGH) + b_ref[...]
    out_ref[0] = h[:, :32]
    out_ref[1] = h[:, 32:]


def _tc_input(x, w, b):
    return pl.pallas_call(
        _tc_input_body,
        grid=(GRID,),
        in_specs=[
            pl.BlockSpec((BN, D_IN), lambda i: (i, 0)),
            pl.BlockSpec((D_IN, H), lambda i: (0, 0)),
            pl.BlockSpec((1, H), lambda i: (0, 0)),
        ],
        out_specs=pl.BlockSpec((NC, BN, 32), lambda i: (0, i, 0)),
        out_shape=jax.ShapeDtypeStruct((NC, N, 32), jnp.float32),
    )(x, w, b)


def _mlp_block(h_ref, a_ref, wa_ref, ba_ref, wb_ref, bb_ref):
    h = jnp.concatenate([h_ref[0], h_ref[1]], axis=1)
    a = jnp.concatenate([a_ref[0], a_ref[1]], axis=1)
    t = 2.0 * h + a
    t = jnp.maximum(jnp.dot(t, wa_ref[...], preferred_element_type=jnp.float32,
                            precision=_HIGH) + ba_ref[...], 0.0)
    return jnp.maximum(jnp.dot(t, wb_ref[...],
                               preferred_element_type=jnp.float32,
                               precision=_HIGH) + bb_ref[...], 0.0)


def _tc_layer_body(h_ref, a_ref, wa_ref, ba_ref, wb_ref, bb_ref, out_ref):
    t = _mlp_block(h_ref, a_ref, wa_ref, ba_ref, wb_ref, bb_ref)
    out_ref[0] = t[:, :32]
    out_ref[1] = t[:, 32:]


_LAYER_IN_SPECS = [
    pl.BlockSpec((NC, BN, 32), lambda i: (0, i, 0)),
    pl.BlockSpec((NC, BN, 32), lambda i: (0, i, 0)),
    pl.BlockSpec((H, H), lambda i: (0, 0)),
    pl.BlockSpec((1, H), lambda i: (0, 0)),
    pl.BlockSpec((H, H), lambda i: (0, 0)),
    pl.BlockSpec((1, H), lambda i: (0, 0)),
]


def _tc_layer(h, agg, wa, ba, wb, bb):
    return pl.pallas_call(
        _tc_layer_body,
        grid=(GRID,),
        in_specs=_LAYER_IN_SPECS,
        out_specs=pl.BlockSpec((NC, BN, 32), lambda i: (0, i, 0)),
        out_shape=jax.ShapeDtypeStruct((NC, N, 32), jnp.float32),
    )(h, agg, wa, ba, wb, bb)


def _tc_last_body(h_ref, a_ref, wa_ref, ba_ref, wb_ref, bb_ref,
                  out_ref, sum_ref):
    i = pl.program_id(0)
    t = _mlp_block(h_ref, a_ref, wa_ref, ba_ref, wb_ref, bb_ref)
    out_ref[0] = t[:, :32]
    out_ref[1] = t[:, 32:]
    part = jnp.sum(t, axis=0, keepdims=True)

    @pl.when(i == 0)
    def _init():
        sum_ref[...] = part

    @pl.when(i != 0)
    def _acc():
        sum_ref[...] += part


def _tc_last(h, agg, wa, ba, wb, bb):
    return pl.pallas_call(
        _tc_last_body,
        grid=(GRID,),
        in_specs=_LAYER_IN_SPECS,
        out_specs=[
            pl.BlockSpec((NC, BN, 32), lambda i: (0, i, 0)),
            pl.BlockSpec((1, H), lambda i: (0, 0)),
        ],
        out_shape=[
            jax.ShapeDtypeStruct((NC, N, 32), jnp.float32),
            jax.ShapeDtypeStruct((1, H), jnp.float32),
        ],
    )(h, agg, wa, ba, wb, bb)


def _tc_head_body(s_ref, w1_ref, b1_ref, w2_ref, b2_ref, out_ref):
    g = s_ref[...] * (1.0 / N)
    z = jnp.maximum(jnp.dot(g, w1_ref[...], preferred_element_type=jnp.float32,
                            precision=_HIGH) + b1_ref[...], 0.0)
    out_ref[...] = jnp.dot(z, w2_ref[...], preferred_element_type=jnp.float32,
                           precision=_HIGH) + b2_ref[...]


def _tc_head(hsum, w1, b1, w2, b2):
    return pl.pallas_call(
        _tc_head_body,
        out_shape=jax.ShapeDtypeStruct((1, OUT), jnp.float32),
    )(hsum, w1, b1, w2, b2)


def _fold_bn(w, b, g, be):
    gs = g * (1.0 / math.sqrt(1.0 + BN_EPS))
    return w * gs[None, :], (b * gs + be)[None, :]


def kernel(x, edge_index, W_in, b_in,
           W0_1, b0_1, g0_1, be0_1, W0_2, b0_2, g0_2, be0_2,
           W1_1, b1_1, g1_1, be1_1, W1_2, b1_2, g1_2, be1_2,
           W2_1, b2_1, g2_1, be2_1, W2_2, b2_2, g2_2, be2_2,
           Wf1, bf1, gf, bef, Wf2, bf2):
    src = edge_index[0]
    dst = edge_index[1]
    pad = EPAD - E
    srcp = jnp.concatenate([src, jnp.zeros((pad,), jnp.int32)])
    src2 = jnp.stack([srcp, srcp + N]).reshape(NC, NS, NCHUNK, CHUNK)
    dst3 = jnp.concatenate(
        [dst, jnp.full((pad,), N, jnp.int32)]).reshape(NS, NCHUNK, CHUNK)

    layers = [
        _fold_bn(W0_1, b0_1, g0_1, be0_1) + _fold_bn(W0_2, b0_2, g0_2, be0_2),
        _fold_bn(W1_1, b1_1, g1_1, be1_1) + _fold_bn(W1_2, b1_2, g1_2, be1_2),
        _fold_bn(W2_1, b2_1, g2_1, be2_1) + _fold_bn(W2_2, b2_2, g2_2, be2_2),
    ]
    wf1, bf1f = _fold_bn(Wf1, bf1, gf, bef)

    h = _tc_input(x, W_in, b_in[None, :])
    hsum = None
    for li, (wa, ba, wb, bb) in enumerate(layers):
        agg = _sc_agg(h.reshape(NC * N, 32), src2, dst3)
        if li < 2:
            h = _tc_layer(h, agg, wa, ba, wb, bb)
        else:
            h, hsum = _tc_last(h, agg, wa, ba, wb, bb)
    return _tc_head(hsum, wf1, bf1f, Wf2, bf2[None, :])
